# baseline (speedup 1.0000x reference)
import functools

import jax
import jax.numpy as jnp
from jax.experimental import pallas as pl
from jax.experimental.pallas import tpu as pltpu
from jax.experimental.pallas import tpu_sc as plsc

_SC_CORES = 2
_SC_SUBCORES = 16
_SC_WORKERS = _SC_CORES * _SC_SUBCORES
_GW = 128


def _sc_mesh():
    return plsc.VectorSubcoreMesh(core_axis_name="core",
                                  subcore_axis_name="subcore")


def _sc_gather(x, idx2, e0, e1):
    Eh = e1 - e0
    N, H = x.shape
    w0 = e0 // _GW

    @functools.partial(
        pl.kernel,
        out_type=jax.ShapeDtypeStruct((Eh, H), x.dtype),
        mesh=_sc_mesh(),
        compiler_params=pltpu.CompilerParams(use_tc_tiling_on_sc=False),
    )
    def k(x_hbm, i_hbm, o_hbm):
        def body(i_vmem, o_vmem):
            pltpu.sync_copy(x_hbm.at[i_vmem.at[0]], o_vmem)

        pltpu.emit_pipeline(
            body,
            grid=(Eh // _GW,),
            in_specs=[pl.BlockSpec((1, _GW), lambda i: (0, i + w0))],
            out_specs=[pl.BlockSpec((_GW, H), lambda i: (i, 0))],
            core_axis_name=("core", "subcore"),
            dimension_semantics=(pltpu.PARALLEL,),
        )(i_hbm, o_hbm)

    return k(x, idx2)


def _sc_segment_sum(msg, dst2, zeros, e0, e1):
    E, H = msg.shape
    N = zeros.shape[0]
    c0 = e0 // _GW
    n_chunks = (e1 - e0) // _GW
    n_rounds = (n_chunks + _SC_WORKERS - 1) // _SC_WORKERS
    rows_per_sub = N // _SC_SUBCORES

    @functools.partial(
        pl.kernel,
        out_type=jax.ShapeDtypeStruct((_SC_CORES, N, H), jnp.float32),
        mesh=_sc_mesh(),
        compiler_params=pltpu.CompilerParams(use_tc_tiling_on_sc=False),
        scratch_types=[
            pltpu.VMEM((_GW,), jnp.int32),
            pltpu.VMEM((_GW, H), jnp.float32),
            pltpu.VMEM_SHARED((N, H), jnp.float32),
        ],
    )
    def k(msg_hbm, dst_hbm, zero_hbm, out_hbm, idx_v, rows_v, agg_sh):
        cid = jax.lax.axis_index("core")
        sid = jax.lax.axis_index("subcore")
        wid = cid * _SC_SUBCORES + sid
        row0 = sid * rows_per_sub
        pltpu.sync_copy(zero_hbm.at[pl.ds(row0, rows_per_sub)],
                        agg_sh.at[pl.ds(row0, rows_per_sub)])
        plsc.subcore_barrier()

        @pl.loop(0, n_rounds)
        def _(r):
            c = wid + r * _SC_WORKERS

            @pl.when(c < n_chunks)
            def _():
                cg = c + c0
                pltpu.sync_copy(dst_hbm.at[0, pl.ds(cg * _GW, _GW)], idx_v)
                pltpu.sync_copy(msg_hbm.at[pl.ds(c * _GW, _GW)], rows_v)
                pltpu.sync_copy(rows_v, agg_sh.at[idx_v], add=True)

        plsc.subcore_barrier()
        pltpu.sync_copy(agg_sh.at[pl.ds(row0, rows_per_sub)],
                        out_hbm.at[cid, pl.ds(row0, rows_per_sub)])

    return k(msg, dst2, zeros)


def _wer_body(ea_ref, wn1_ref, bn1_ref, wn2r_ref, bn2r_ref, o_ref):
    z = jnp.dot(ea_ref[...], wn1_ref[...],
                preferred_element_type=jnp.float32) + bn1_ref[...]
    z = z * jax.nn.sigmoid(z)
    wer = jnp.dot(z.astype(jnp.bfloat16), wn2r_ref[...],
                  preferred_element_type=jnp.float32) + bn2r_ref[...]
    o_ref[...] = wer.astype(jnp.bfloat16)


def _tc_wer(edge_attr, Wn1, bn1, Wn2, bn2):
    E, A = edge_attr.shape
    M = Wn1.shape[1]
    H = int((Wn2.shape[1]) ** 0.5)
    wn2r = Wn2.reshape(M, H, H).transpose(0, 2, 1).reshape(M, H * H)
    bn2r = bn2.reshape(H, H).T.reshape(1, H * H)
    EB = 2000
    return pl.pallas_call(
        _wer_body,
        grid=(E // EB,),
        in_specs=[
            pl.BlockSpec((EB, A), lambda i: (i, 0)),
            pl.BlockSpec(Wn1.shape, lambda i: (0, 0)),
            pl.BlockSpec((1, M), lambda i: (0, 0)),
            pl.BlockSpec((M, H * H), lambda i: (0, 0)),
            pl.BlockSpec((1, H * H), lambda i: (0, 0)),
        ],
        out_specs=pl.BlockSpec((EB, H * H), lambda i: (i, 0)),
        out_shape=jax.ShapeDtypeStruct((E, H * H), jnp.bfloat16),
    )(edge_attr, Wn1, bn1.reshape(1, M), wn2r.astype(jnp.bfloat16), bn2r)


def _msg_body(wer_ref, xg_ref, g_ref, o_ref):
    H = xg_ref.shape[1]
    xg = xg_ref[...]
    xt = jnp.concatenate([xg] * H, axis=1)
    p = wer_ref[...] * xt
    o_ref[...] = jnp.dot(p, g_ref[...], preferred_element_type=jnp.float32)


def _tc_messages(wer, xg, e0):
    Eh, H = xg.shape
    g = jnp.repeat(jnp.eye(H, dtype=jnp.bfloat16), H, axis=0)
    EB = 2000
    b0 = e0 // EB
    return pl.pallas_call(
        _msg_body,
        grid=(Eh // EB,),
        in_specs=[
            pl.BlockSpec((EB, H * H), lambda i: (i + b0, 0)),
            pl.BlockSpec((EB, H), lambda i: (i, 0)),
            pl.BlockSpec((H * H, H), lambda i: (0, 0)),
        ],

        out_specs=pl.BlockSpec((EB, H), lambda i: (i, 0)),
        out_shape=jax.ShapeDtypeStruct((Eh, H), jnp.float32),
    )(wer, xg, g)


def _gru_body(pa_ref, pb_ref, x_ref, h_ref, root_ref, cb_ref, wih_ref,
              bih_ref, whh_ref, bhh_ref, o_ref, ob_ref):
    H = x_ref.shape[1]
    agg = (pa_ref[0] + pa_ref[1]) + (pb_ref[0] + pb_ref[1])
    mm = agg + jnp.dot(x_ref[...], root_ref[...],
                       preferred_element_type=jnp.float32) + cb_ref[...]
    m = mm * jax.nn.sigmoid(mm)
    gi = jnp.dot(m, wih_ref[...], preferred_element_type=jnp.float32) + bih_ref[...]
    gh = jnp.dot(h_ref[...], whh_ref[...], preferred_element_type=jnp.float32) + bhh_ref[...]
    r = jax.nn.sigmoid(gi[:, :H] + gh[:, :H])
    z = jax.nn.sigmoid(gi[:, H:2 * H] + gh[:, H:2 * H])
    n = jnp.tanh(gi[:, 2 * H:] + r * gh[:, 2 * H:])
    hn = (1.0 - z) * n + z * h_ref[...]
    o_ref[...] = hn
    ob_ref[...] = hn.astype(jnp.bfloat16)


def _tc_gru_update(pa, pb, x, h, root, conv_bias, W_ih, b_ih, W_hh, b_hh):
    N, H = x.shape
    return pl.pallas_call(
        _gru_body,
        out_shape=[jax.ShapeDtypeStruct((N, H), jnp.float32),
                   jax.ShapeDtypeStruct((N, H), jnp.bfloat16)],
    )(pa, pb, x, h, root, conv_bias.reshape(1, H), W_ih,
      b_ih.reshape(1, 3 * H), W_hh, b_hh.reshape(1, 3 * H))


def _set2set_body(x_ref, b_ref, wli_ref, bli_ref, wlh_ref, blh_ref, wo1_ref,
                  bo1_ref, wo2_ref, bo2_ref, o_ref):
    N, H = x_ref.shape
    B = 16
    x = x_ref[...]
    onehot = (b_ref[...] == jax.lax.broadcasted_iota(jnp.int32, (N, B), 1))
    onehot = onehot.astype(jnp.float32)
    q_star = jnp.zeros((B, 2 * H), jnp.float32)
    hl = jnp.zeros((B, H), jnp.float32)
    cl = jnp.zeros((B, H), jnp.float32)
    for _ in range(3):
        gates = (jnp.dot(q_star, wli_ref[...], preferred_element_type=jnp.float32)
                 + bli_ref[...]
                 + jnp.dot(hl, wlh_ref[...], preferred_element_type=jnp.float32)
                 + blh_ref[...])
        gi = jax.nn.sigmoid(gates[:, :H])
        gf = jax.nn.sigmoid(gates[:, H:2 * H])
        gg = jnp.tanh(gates[:, 2 * H:3 * H])
        go = jax.nn.sigmoid(gates[:, 3 * H:])
        cl = gf * cl + gi * gg
        hl = go * jnp.tanh(cl)
        q = hl
        mq = jnp.dot(x, q.T, preferred_element_type=jnp.float32)
        e = jnp.sum(mq * onehot, axis=1, keepdims=True)
        emax = jnp.max(jnp.where(onehot > 0, e, -jnp.inf), axis=0)
        emax = jnp.where(emax > -jnp.inf, emax, 0.0)
        ee = jnp.exp(e - jnp.sum(onehot * emax[None, :], axis=1, keepdims=True))
        esum = jnp.sum(onehot * ee, axis=0)
        a = ee / (jnp.sum(onehot * esum[None, :], axis=1, keepdims=True) + 1e-16)
        r = jax.lax.dot_general(onehot * a, x, (((0,), (0,)), ((), ())),
                                preferred_element_type=jnp.float32)
        q_star = jnp.concatenate([q, r], axis=1)
    u = jnp.dot(q_star, wo1_ref[...], preferred_element_type=jnp.float32) + bo1_ref[...]
    u = u * jax.nn.sigmoid(u)
    o_ref[...] = jnp.dot(u, wo2_ref[...], preferred_element_type=jnp.float32) + bo2_ref[...]


def _tc_set2set(x, batch, Wl_i, bl_i, Wl_h, bl_h, Wo1, bo1, Wo2, bo2):
    N, H = x.shape
    out = pl.pallas_call(
        _set2set_body,
        out_shape=jax.ShapeDtypeStruct((16, 1), jnp.float32),
    )(x, batch.astype(jnp.int32).reshape(N, 1), Wl_i,
      bl_i.reshape(1, -1), Wl_h, bl_h.reshape(1, -1), Wo1,
      bo1.reshape(1, -1), Wo2, bo2.reshape(1, -1))
    return out.reshape(-1)


def _first_layer_body(x_ref, w_ref, b_ref, o_ref, ob_ref):
    acc = jnp.dot(x_ref[...], w_ref[...], preferred_element_type=jnp.float32)
    acc = acc + b_ref[...]
    res = acc * jax.nn.sigmoid(acc)
    o_ref[...] = res
    ob_ref[...] = res.astype(jnp.bfloat16)


def kernel(x, edge_index, edge_attr, batch, W1, b1, Wn1, bn1, Wn2, bn2, root,
           conv_bias, W_ih, b_ih, W_hh, b_hh, Wl_i, bl_i, Wl_h, bl_h, Wo1,
           bo1, Wo2, bo2):
    silu = jax.nn.silu
    N, _ = x.shape
    H = root.shape[0]
    E = edge_index.shape[1]
    B = 16
    src = edge_index[0]
    dst = edge_index[1]

    x, x_bf = pl.pallas_call(
        _first_layer_body,
        out_shape=[jax.ShapeDtypeStruct((N, H), jnp.float32),
                   jax.ShapeDtypeStruct((N, H), jnp.bfloat16)],
    )(x, W1, b1[None, :])
    h = x

    src2 = src.astype(jnp.int32).reshape(1, E)
    dst2 = dst.astype(jnp.int32).reshape(1, E)
    zeros = jnp.zeros((N, H), jnp.float32)
    Em = E // 2
    wer = _tc_wer(edge_attr, Wn1, bn1, Wn2, bn2)
    for _ in range(4):
        xg_a = _sc_gather(x_bf, src2, 0, Em)
        xg_b = _sc_gather(x_bf, src2, Em, E)
        msg_a = _tc_messages(wer, xg_a, 0)
        msg_b = _tc_messages(wer, xg_b, Em)
        part_a = _sc_segment_sum(msg_a, dst2, zeros, 0, Em)
        part_b = _sc_segment_sum(msg_b, dst2, zeros, Em, E)
        h, x_bf = _tc_gru_update(part_a, part_b, x, h, root, conv_bias, W_ih,
                                 b_ih, W_hh, b_hh)
        x = h

    return _tc_set2set(x, batch, Wl_i, bl_i, Wl_h, bl_h, Wo1, bo1, Wo2, bo2)


# speedup vs baseline: 1.0263x; 1.0263x over previous
import functools

import jax
import jax.numpy as jnp
from jax.experimental import pallas as pl
from jax.experimental.pallas import tpu as pltpu
from jax.experimental.pallas import tpu_sc as plsc

_SC_CORES = 2
_SC_SUBCORES = 16
_SC_WORKERS = _SC_CORES * _SC_SUBCORES
_GW = 128


def _sc_mesh():
    return plsc.VectorSubcoreMesh(core_axis_name="core",
                                  subcore_axis_name="subcore")


def _sc_gather(x, idx2, e0, e1):
    Eh = e1 - e0
    N, H = x.shape
    w0 = e0 // _GW

    @functools.partial(
        pl.kernel,
        out_type=jax.ShapeDtypeStruct((Eh, H), x.dtype),
        mesh=_sc_mesh(),
        compiler_params=pltpu.CompilerParams(use_tc_tiling_on_sc=False),
    )
    def k(x_hbm, i_hbm, o_hbm):
        def body(i_vmem, o_vmem):
            pltpu.sync_copy(x_hbm.at[i_vmem.at[0]], o_vmem)

        pltpu.emit_pipeline(
            body,
            grid=(Eh // _GW,),
            in_specs=[pl.BlockSpec((1, _GW), lambda i: (0, i + w0))],
            out_specs=[pl.BlockSpec((_GW, H), lambda i: (i, 0))],
            core_axis_name=("core", "subcore"),
            dimension_semantics=(pltpu.PARALLEL,),
        )(i_hbm, o_hbm)

    return k(x, idx2)


def _sc_segment_sum(msg, dst2, zeros, e0, e1):
    E, H = msg.shape
    N = zeros.shape[0]
    c0 = e0 // _GW
    n_chunks = (e1 - e0) // _GW
    n_rounds = (n_chunks + _SC_WORKERS - 1) // _SC_WORKERS
    rows_per_sub = N // _SC_SUBCORES

    @functools.partial(
        pl.kernel,
        out_type=jax.ShapeDtypeStruct((_SC_CORES, N, H), jnp.float32),
        mesh=_sc_mesh(),
        compiler_params=pltpu.CompilerParams(use_tc_tiling_on_sc=False),
        scratch_types=[
            pltpu.VMEM((_GW,), jnp.int32),
            pltpu.VMEM((_GW, H), jnp.float32),
            pltpu.VMEM_SHARED((N, H), jnp.float32),
        ],
    )
    def k(msg_hbm, dst_hbm, zero_hbm, out_hbm, idx_v, rows_v, agg_sh):
        cid = jax.lax.axis_index("core")
        sid = jax.lax.axis_index("subcore")
        wid = cid * _SC_SUBCORES + sid
        row0 = sid * rows_per_sub
        pltpu.sync_copy(zero_hbm.at[pl.ds(row0, rows_per_sub)],
                        agg_sh.at[pl.ds(row0, rows_per_sub)])
        plsc.subcore_barrier()

        @pl.loop(0, n_rounds)
        def _(r):
            c = wid + r * _SC_WORKERS

            @pl.when(c < n_chunks)
            def _():
                cg = c + c0
                pltpu.sync_copy(dst_hbm.at[0, pl.ds(cg * _GW, _GW)], idx_v)
                pltpu.sync_copy(msg_hbm.at[pl.ds(c * _GW, _GW)], rows_v)
                pltpu.sync_copy(rows_v, agg_sh.at[idx_v], add=True)

        plsc.subcore_barrier()
        pltpu.sync_copy(agg_sh.at[pl.ds(row0, rows_per_sub)],
                        out_hbm.at[cid, pl.ds(row0, rows_per_sub)])

    return k(msg, dst2, zeros)


def _wer_body(ea_ref, wn1_ref, bn1_ref, wn2r_ref, bn2r_ref, o_ref):
    z = jnp.dot(ea_ref[...], wn1_ref[...],
                preferred_element_type=jnp.float32) + bn1_ref[...]
    z = z * jax.nn.sigmoid(z)
    wer = jnp.dot(z.astype(jnp.bfloat16), wn2r_ref[...],
                  preferred_element_type=jnp.float32) + bn2r_ref[...]
    o_ref[...] = wer.astype(jnp.bfloat16)


def _tc_wer(edge_attr, Wn1, bn1, Wn2, bn2):
    E, A = edge_attr.shape
    M = Wn1.shape[1]
    H = int((Wn2.shape[1]) ** 0.5)
    wn2r = Wn2.reshape(M, H, H).transpose(0, 2, 1).reshape(M, H * H)
    bn2r = bn2.reshape(H, H).T.reshape(1, H * H)
    EB = 2000
    return pl.pallas_call(
        _wer_body,
        grid=(E // EB,),
        in_specs=[
            pl.BlockSpec((EB, A), lambda i: (i, 0)),
            pl.BlockSpec(Wn1.shape, lambda i: (0, 0)),
            pl.BlockSpec((1, M), lambda i: (0, 0)),
            pl.BlockSpec((M, H * H), lambda i: (0, 0)),
            pl.BlockSpec((1, H * H), lambda i: (0, 0)),
        ],
        out_specs=pl.BlockSpec((EB, H * H), lambda i: (i, 0)),
        out_shape=jax.ShapeDtypeStruct((E, H * H), jnp.bfloat16),
    )(edge_attr, Wn1, bn1.reshape(1, M), wn2r.astype(jnp.bfloat16), bn2r)


def _msg_body(wer_ref, xg_ref, g_ref, o_ref):
    H = xg_ref.shape[1]
    xg = xg_ref[...].astype(jnp.bfloat16)
    xt = jnp.concatenate([xg] * H, axis=1)
    p = wer_ref[...] * xt
    o_ref[...] = jnp.dot(p, g_ref[...], preferred_element_type=jnp.float32)


def _tc_messages(wer, xg, e0):
    Eh, H = xg.shape
    g = jnp.repeat(jnp.eye(H, dtype=jnp.bfloat16), H, axis=0)
    EB = 2000
    b0 = e0 // EB
    return pl.pallas_call(
        _msg_body,
        grid=(Eh // EB,),
        in_specs=[
            pl.BlockSpec((EB, H * H), lambda i: (i + b0, 0)),
            pl.BlockSpec((EB, H), lambda i: (i, 0)),
            pl.BlockSpec((H * H, H), lambda i: (0, 0)),
        ],
        out_specs=pl.BlockSpec((EB, H), lambda i: (i, 0)),
        out_shape=jax.ShapeDtypeStruct((Eh, H), jnp.float32),
    )(wer, xg, g)


def _gru_body(pa_ref, pb_ref, x_ref, h_ref, root_ref, cb_ref, wih_ref,
              bih_ref, whh_ref, bhh_ref, o_ref):
    H = x_ref.shape[1]
    agg = (pa_ref[0] + pa_ref[1]) + (pb_ref[0] + pb_ref[1])
    mm = agg + jnp.dot(x_ref[...], root_ref[...],
                       preferred_element_type=jnp.float32) + cb_ref[...]
    m = mm * jax.nn.sigmoid(mm)
    gi = jnp.dot(m, wih_ref[...], preferred_element_type=jnp.float32) + bih_ref[...]
    gh = jnp.dot(h_ref[...], whh_ref[...], preferred_element_type=jnp.float32) + bhh_ref[...]
    r = jax.nn.sigmoid(gi[:, :H] + gh[:, :H])
    z = jax.nn.sigmoid(gi[:, H:2 * H] + gh[:, H:2 * H])
    n = jnp.tanh(gi[:, 2 * H:] + r * gh[:, 2 * H:])
    o_ref[...] = (1.0 - z) * n + z * h_ref[...]


def _tc_gru_update(pa, pb, x, h, root, conv_bias, W_ih, b_ih, W_hh, b_hh):
    N, H = x.shape
    return pl.pallas_call(
        _gru_body,
        out_shape=jax.ShapeDtypeStruct((N, H), jnp.float32),
    )(pa, pb, x, h, root, conv_bias.reshape(1, H), W_ih,
      b_ih.reshape(1, 3 * H), W_hh, b_hh.reshape(1, 3 * H))


def _set2set_body(x_ref, b_ref, wli_ref, bli_ref, wlh_ref, blh_ref, wo1_ref,
                  bo1_ref, wo2_ref, bo2_ref, o_ref):
    N, H = x_ref.shape
    B = 16
    x = x_ref[...]
    onehot = (b_ref[...] == jax.lax.broadcasted_iota(jnp.int32, (N, B), 1))
    onehot = onehot.astype(jnp.float32)
    q_star = jnp.zeros((B, 2 * H), jnp.float32)
    hl = jnp.zeros((B, H), jnp.float32)
    cl = jnp.zeros((B, H), jnp.float32)
    for _ in range(3):
        gates = (jnp.dot(q_star, wli_ref[...], preferred_element_type=jnp.float32)
                 + bli_ref[...]
                 + jnp.dot(hl, wlh_ref[...], preferred_element_type=jnp.float32)
                 + blh_ref[...])
        gi = jax.nn.sigmoid(gates[:, :H])
        gf = jax.nn.sigmoid(gates[:, H:2 * H])
        gg = jnp.tanh(gates[:, 2 * H:3 * H])
        go = jax.nn.sigmoid(gates[:, 3 * H:])
        cl = gf * cl + gi * gg
        hl = go * jnp.tanh(cl)
        q = hl
        mq = jnp.dot(x, q.T, preferred_element_type=jnp.float32)
        e = jnp.sum(mq * onehot, axis=1, keepdims=True)
        emax = jnp.max(jnp.where(onehot > 0, e, -jnp.inf), axis=0)
        emax = jnp.where(emax > -jnp.inf, emax, 0.0)
        ee = jnp.exp(e - jnp.sum(onehot * emax[None, :], axis=1, keepdims=True))
        esum = jnp.sum(onehot * ee, axis=0)
        a = ee / (jnp.sum(onehot * esum[None, :], axis=1, keepdims=True) + 1e-16)
        r = jax.lax.dot_general(onehot * a, x, (((0,), (0,)), ((), ())),
                                preferred_element_type=jnp.float32)
        q_star = jnp.concatenate([q, r], axis=1)
    u = jnp.dot(q_star, wo1_ref[...], preferred_element_type=jnp.float32) + bo1_ref[...]
    u = u * jax.nn.sigmoid(u)
    o_ref[...] = jnp.dot(u, wo2_ref[...], preferred_element_type=jnp.float32) + bo2_ref[...]


def _tc_set2set(x, batch, Wl_i, bl_i, Wl_h, bl_h, Wo1, bo1, Wo2, bo2):
    N, H = x.shape
    out = pl.pallas_call(
        _set2set_body,
        out_shape=jax.ShapeDtypeStruct((16, 1), jnp.float32),
    )(x, batch.astype(jnp.int32).reshape(N, 1), Wl_i,
      bl_i.reshape(1, -1), Wl_h, bl_h.reshape(1, -1), Wo1,
      bo1.reshape(1, -1), Wo2, bo2.reshape(1, -1))
    return out.reshape(-1)


def _first_layer_body(x_ref, w_ref, b_ref, o_ref):
    acc = jnp.dot(x_ref[...], w_ref[...], preferred_element_type=jnp.float32)
    acc = acc + b_ref[...]
    o_ref[...] = acc * jax.nn.sigmoid(acc)


def kernel(x, edge_index, edge_attr, batch, W1, b1, Wn1, bn1, Wn2, bn2, root,
           conv_bias, W_ih, b_ih, W_hh, b_hh, Wl_i, bl_i, Wl_h, bl_h, Wo1,
           bo1, Wo2, bo2):
    silu = jax.nn.silu
    N, _ = x.shape
    H = root.shape[0]
    E = edge_index.shape[1]
    B = 16
    src = edge_index[0]
    dst = edge_index[1]

    x = pl.pallas_call(
        _first_layer_body,
        out_shape=jax.ShapeDtypeStruct((N, H), jnp.float32),
    )(x, W1, b1[None, :])
    h = x

    src2 = src.astype(jnp.int32).reshape(1, E)
    dst2 = dst.astype(jnp.int32).reshape(1, E)
    zeros = jnp.zeros((N, H), jnp.float32)
    Em = E // 2
    wer = _tc_wer(edge_attr, Wn1, bn1, Wn2, bn2)
    for _ in range(4):
        xg_a = _sc_gather(x, src2, 0, Em)
        xg_b = _sc_gather(x, src2, Em, E)
        msg_a = _tc_messages(wer, xg_a, 0)
        msg_b = _tc_messages(wer, xg_b, Em)
        part_a = _sc_segment_sum(msg_a, dst2, zeros, 0, Em)
        part_b = _sc_segment_sum(msg_b, dst2, zeros, Em, E)
        h = _tc_gru_update(part_a, part_b, x, h, root, conv_bias, W_ih, b_ih,
                           W_hh, b_hh)
        x = h

    return _tc_set2set(x, batch, Wl_i, bl_i, Wl_h, bl_h, Wo1, bo1, Wo2, bo2)
